# Initial kernel scaffold; baseline (speedup 1.0000x reference)
import jax
import jax.numpy as jnp
from jax.experimental import pallas as pl


def kernel(edge_index, emb, W1, b1, W2, b2):
    raise NotImplementedError("write your pallas kernel here")


# speedup vs baseline: 19.4593x; 19.4593x over previous
import functools

import jax
import jax.numpy as jnp
from jax import lax
from jax.experimental import pallas as pl
from jax.experimental.pallas import tpu as pltpu
from jax.experimental.pallas import tpu_sc as plsc

N = 10000
E = 320000
D_EMB = 128
D_H1 = 64
D_H2 = 32

NP = 10240
NR = NP // 128
NC = 2
NS = 16
NW = NC * NS
L = 128
RW = 80
E_PAD = NW * RW * L
SRPS = NP // NS

_SC_MESH = dict(core_axis_name="c", subcore_axis_name="s",
                num_cores=NC, num_subcores=NS)
_CP = pltpu.CompilerParams(needs_layout_passes=False)


@functools.cache
def _get_deg_kernel():
    @functools.partial(
        pl.kernel,
        out_type=jax.ShapeDtypeStruct((NW, NR, 128), jnp.float32),
        mesh=plsc.VectorSubcoreMesh(**_SC_MESH),
        compiler_params=_CP,
        scratch_types=[
            pltpu.VMEM((RW, L), jnp.int32),
            pltpu.VMEM((NR, 128), jnp.float32),
        ],
    )
    def deg(dst_hbm, out_hbm, didx, cnt):
        c = lax.axis_index("c")
        s = lax.axis_index("s")
        wid = c * NS + s
        pltpu.sync_copy(dst_hbm.at[pl.ds(wid * RW, RW)], didx)
        zero = jnp.zeros((16,), jnp.float32)

        def zb(r, carry):
            for k in range(8):
                cnt[r, pl.ds(k * 16, 16)] = zero
            return carry

        lax.fori_loop(0, NR, zb, 0)
        ones16 = jnp.ones((16,), jnp.float32)

        def body(j, carry):
            def inner(k, carry2):
                v = didx[j, pl.ds(k * 16, 16)]
                plsc.addupdate_scatter(cnt, [v >> 7, v & 127], ones16)
                return carry2

            lax.fori_loop(0, 8, inner, 0)
            return carry

        lax.fori_loop(0, RW, body, 0)
        plsc.subcore_barrier()
        pltpu.sync_copy(cnt, out_hbm.at[wid])

    return deg


@functools.cache
def _get_agg_kernel():
    @functools.partial(
        pl.kernel,
        out_type=jax.ShapeDtypeStruct((NC, NP, 128), jnp.float32),
        mesh=plsc.VectorSubcoreMesh(**_SC_MESH),
        compiler_params=_CP,
        scratch_types=[
            pltpu.VMEM((RW, L), jnp.int32),
            pltpu.VMEM((RW, L), jnp.int32),
            pltpu.VMEM((L, 128), jnp.float32),
            pltpu.VMEM_SHARED((NP, 128), jnp.float32),
        ],
    )
    def agg(g_hbm, src_hbm, dst_hbm, out_hbm, sidx, didx, rows, acc):
        c = lax.axis_index("c")
        s = lax.axis_index("s")
        wid = c * NS + s
        zero = jnp.zeros((16,), jnp.float32)

        def zb(r, carry):
            for k in range(8):
                rows[r, pl.ds(k * 16, 16)] = zero
            return carry

        lax.fori_loop(0, L, zb, 0)
        for i in range(SRPS // L):
            pltpu.sync_copy(rows, acc.at[pl.ds(s * SRPS + i * L, L)])
        pltpu.sync_copy(src_hbm.at[pl.ds(wid * RW, RW)], sidx)
        pltpu.sync_copy(dst_hbm.at[pl.ds(wid * RW, RW)], didx)
        plsc.subcore_barrier()

        def body(j, carry):
            pltpu.sync_copy(g_hbm.at[sidx.at[j]], rows)
            pltpu.sync_copy(rows, acc.at[didx.at[j]], add=True)
            return carry

        lax.fori_loop(0, RW, body, 0)
        plsc.subcore_barrier()
        pltpu.sync_copy(acc.at[pl.ds(s * SRPS, SRPS)],
                        out_hbm.at[c, pl.ds(s * SRPS, SRPS)])

    return agg


_DP = 8
_HB = RW // 2


@functools.cache
def _get_decode_kernel():
    @functools.partial(
        pl.kernel,
        out_type=jax.ShapeDtypeStruct((E_PAD // L, 128), jnp.float32),
        mesh=plsc.VectorSubcoreMesh(**_SC_MESH),
        compiler_params=_CP,
        scratch_types=[
            pltpu.VMEM((_HB, L), jnp.int32),
            pltpu.VMEM((_HB, L), jnp.int32),
            pltpu.VMEM((_DP, NP), jnp.float32),
            pltpu.VMEM((_HB, L), jnp.float32),
        ],
    )
    def decode(zt_hbm, src_hbm, dst_hbm, out_hbm, idx0, idx1, planes, acc):
        c = lax.axis_index("c")
        s = lax.axis_index("s")
        wid = c * NS + s
        zero = jnp.zeros((16,), jnp.float32)

        for h in range(2):
            rbase = wid * RW + h * _HB
            pltpu.sync_copy(src_hbm.at[pl.ds(rbase, _HB)], idx0)
            pltpu.sync_copy(dst_hbm.at[pl.ds(rbase, _HB)], idx1)

            def zb(r, carry):
                for k in range(8):
                    acc[r, pl.ds(k * 16, 16)] = zero
                return carry

            lax.fori_loop(0, _HB, zb, 0)
            for p in range(D_H2 // _DP):
                pltpu.sync_copy(zt_hbm.at[pl.ds(p * _DP, _DP)], planes)

                def body(j, carry):
                    def inner(k, carry2):
                        a0 = idx0[j, pl.ds(k * 16, 16)]
                        a1 = idx1[j, pl.ds(k * 16, 16)]
                        t = acc[j, pl.ds(k * 16, 16)]
                        for d in range(_DP):
                            dv = jnp.full((16,), d, jnp.int32)
                            z0 = plsc.load_gather(planes, [dv, a0])
                            z1 = plsc.load_gather(planes, [dv, a1])
                            t = t + z0 * z1
                        acc[j, pl.ds(k * 16, 16)] = t
                        return carry2

                    lax.fori_loop(0, 8, inner, 0)
                    return carry

                lax.fori_loop(0, _HB, body, 0)

            def sg(j, carry):
                for k in range(8):
                    t = acc[j, pl.ds(k * 16, 16)]
                    acc[j, pl.ds(k * 16, 16)] = 1.0 / (1.0 + jnp.exp(-t))
                return carry

            lax.fori_loop(0, _HB, sg, 0)
            plsc.subcore_barrier()
            pltpu.sync_copy(acc, out_hbm.at[pl.ds(rbase, _HB)])

    return decode


def _dinv128(deg80):
    dinv = lax.rsqrt(deg80)
    tiled = jnp.broadcast_to(dinv[:, None, :], (NR, 128, 128)).reshape(NP, 128)
    row = lax.broadcasted_iota(jnp.int32, (NP, 128), 0)
    col = lax.broadcasted_iota(jnp.int32, (NP, 128), 1)
    masked = jnp.where((row % 128) == col, tiled, 0.0)
    ones = jnp.ones((128, 128), jnp.float32)
    return jnp.dot(masked, ones, preferred_element_type=jnp.float32)


def _enc1_body(degp_ref, emb_ref, w1_ref, g1_ref, dinv_ref):
    deg = jnp.sum(degp_ref[...], axis=0) + 1.0
    dinv = _dinv128(deg)
    h1 = jnp.dot(emb_ref[...], w1_ref[...], preferred_element_type=jnp.float32)
    g1 = h1 * dinv[:, :D_H1]
    g1_ref[...] = jnp.concatenate([g1, g1], axis=1)
    dinv_ref[...] = dinv


_enc1 = pl.pallas_call(
    _enc1_body,
    out_shape=(
        jax.ShapeDtypeStruct((NP, 128), jnp.float32),
        jax.ShapeDtypeStruct((NP, 128), jnp.float32),
    ),
)


def _enc2_body(p_ref, g1_ref, dinv_ref, b1_ref, w2_ref, g2_ref):
    dinv = dinv_ref[...]
    psum = p_ref[0, :, :D_H1] + p_ref[1, :, :D_H1] + g1_ref[:, :D_H1]
    x1 = jnp.maximum(psum * dinv[:, :D_H1] + b1_ref[...], 0.0)
    h2 = jnp.dot(x1, w2_ref[...], preferred_element_type=jnp.float32)
    g2 = h2 * dinv[:, :D_H2]
    g2_ref[...] = jnp.concatenate([g2, g2, g2, g2], axis=1)


_enc2 = pl.pallas_call(
    _enc2_body,
    out_shape=jax.ShapeDtypeStruct((NP, 128), jnp.float32),
)


def _enc3_body(q_ref, g2_ref, dinv_ref, b2_ref, zt_ref):
    qsum = q_ref[0, :, :D_H2] + q_ref[1, :, :D_H2] + g2_ref[:, :D_H2]
    z = qsum * dinv_ref[:, :D_H2] + b2_ref[...]
    row = lax.broadcasted_iota(jnp.int32, (128, 128), 0)
    col = lax.broadcasted_iota(jnp.int32, (128, 128), 1)
    eye = jnp.where(row == col, 1.0, 0.0)
    dn = (((0,), (0,)), ((), ()))
    for b in range(NR):
        blk = z[b * 128:(b + 1) * 128, :]
        zt_ref[:, b * 128:(b + 1) * 128] = lax.dot_general(
            blk, eye, dn, preferred_element_type=jnp.float32)


_enc3 = pl.pallas_call(
    _enc3_body,
    out_shape=jax.ShapeDtypeStruct((D_H2, NP), jnp.float32),
)


def kernel(edge_index, emb, W1, b1, W2, b2):
    n_fake = E_PAD - E
    fake = N + jnp.arange(n_fake, dtype=jnp.int32) % (NP - N)
    src2 = jnp.concatenate([edge_index[0], fake]).reshape(E_PAD // L, L)
    dst2 = jnp.concatenate([edge_index[1], fake]).reshape(E_PAD // L, L)
    emb_pad = jnp.pad(emb, ((0, NP - N), (0, 0)))

    degp = _get_deg_kernel()(dst2)
    g1dup, dinv = _enc1(degp, emb_pad, W1)
    P = _get_agg_kernel()(g1dup, src2, dst2)
    g2dup = _enc2(P, g1dup, dinv, b1.reshape(1, D_H1), W2)
    Q = _get_agg_kernel()(g2dup, src2, dst2)
    zt = _enc3(Q, g2dup, dinv, b2.reshape(1, D_H2))
    probs2d = _get_decode_kernel()(zt, src2, dst2)
    return probs2d.reshape(E_PAD)[:E]
